# Initial kernel scaffold; baseline (speedup 1.0000x reference)
import jax
import jax.numpy as jnp
from jax import lax
from jax.experimental import pallas as pl
from jax.experimental.pallas import tpu as pltpu
from jax.experimental.pallas import tpu_sc as plsc

N_DENSE = 13
N_SPARSE = 26
VOCAB = 100000
EMBED = 16
B = 16384
H1, H2, OUT = 128, 64, 1
D_IN = EMBED + N_DENSE

NC, NS = 2, 16
NW = NC * NS
SPW = B // NW
ROWS_PW = SPW * N_SPARSE
GATHER_ROWS = 128
NG = ROWS_PW // GATHER_ROWS
CH_G = 13
SPC = CH_G * GATHER_ROWS // N_SPARSE
NCHUNK = NG // CH_G


def _sc_pool_body(idx_hbm, table_hbm, inter_hbm,
                  idx_v, rows0, rows1, out_v, sem0, sem1):
    cid = lax.axis_index("c")
    sid = lax.axis_index("s")
    wid = sid * NC + cid

    pltpu.sync_copy(idx_hbm.at[wid], idx_v)

    def fire(c, rows, sem):
        descs = []
        for j in range(CH_G):
            d = pltpu.async_copy(
                table_hbm.at[idx_v.at[c * CH_G + j]],
                rows.at[pl.ds(j * GATHER_ROWS, GATHER_ROWS)],
                sem)
            descs.append(d)
        return descs

    def accumulate(c, rows):
        def body(b, carry):
            base = b * N_SPARSE
            v = rows[base]
            s = v
            q = v * v
            for f in range(1, N_SPARSE):
                v = rows[base + f]
                s = s + v
                q = q + v * v
            out_v[c * SPC + b] = 0.5 * (s * s - q)
            return carry
        lax.fori_loop(0, SPC, body, 0)

    bufs = (rows0, rows1)
    sems = (sem0, sem1)
    descs = fire(0, bufs[0], sems[0])
    for c in range(NCHUNK):
        nxt = None
        if c + 1 < NCHUNK:
            nxt = fire(c + 1, bufs[(c + 1) % 2], sems[(c + 1) % 2])
        for d in descs:
            d.wait()
        accumulate(c, bufs[c % 2])
        descs = nxt

    pltpu.sync_copy(out_v, inter_hbm.at[pl.ds(wid * SPW, SPW)])


@jax.jit
def _sc_pool(idx3d, table2d):
    mesh = plsc.VectorSubcoreMesh(core_axis_name="c", subcore_axis_name="s")
    return pl.kernel(
        _sc_pool_body,
        out_type=jax.ShapeDtypeStruct((B, EMBED), jnp.float32),
        mesh=mesh,
        scratch_types=[
            pltpu.VMEM((NG, GATHER_ROWS), jnp.int32),
            pltpu.VMEM((CH_G * GATHER_ROWS, EMBED), jnp.float32),
            pltpu.VMEM((CH_G * GATHER_ROWS, EMBED), jnp.float32),
            pltpu.VMEM((SPW, EMBED), jnp.float32),
            pltpu.SemaphoreType.DMA,
            pltpu.SemaphoreType.DMA,
        ],
    )(idx3d, table2d)


BS = 2048


def _mlp_body(x_ref, g_ref, be_ref, mm_ref, mv_ref,
              w1_ref, b1_ref, w2_ref, b2_ref, w3_ref, b3_ref, o_ref):
    x = x_ref[...]
    inv = lax.rsqrt(mv_ref[...] + 1e-3)
    x = (x - mm_ref[...]) * inv * g_ref[...] + be_ref[...]
    h = jnp.dot(x, w1_ref[...], preferred_element_type=jnp.float32) + b1_ref[...]
    h = jnp.maximum(h, 0.0)
    h = jnp.dot(h, w2_ref[...], preferred_element_type=jnp.float32) + b2_ref[...]
    h = jnp.maximum(h, 0.0)
    o = jnp.dot(h, w3_ref[...], preferred_element_type=jnp.float32) + b3_ref[...]
    o_ref[...] = jax.nn.sigmoid(o)


@jax.jit
def _mlp(x, gamma, beta, mm, mv, W1, b1, W2, b2, W3, b3):
    full = lambda shape: pl.BlockSpec(shape, lambda i: (0, 0))
    return pl.pallas_call(
        _mlp_body,
        grid=(B // BS,),
        in_specs=[
            pl.BlockSpec((BS, D_IN), lambda i: (i, 0)),
            full((1, D_IN)), full((1, D_IN)), full((1, D_IN)), full((1, D_IN)),
            full((D_IN, H1)), full((1, H1)),
            full((H1, H2)), full((1, H2)),
            full((H2, OUT)), full((1, OUT)),
        ],
        out_specs=pl.BlockSpec((BS, OUT), lambda i: (i, 0)),
        out_shape=jax.ShapeDtypeStruct((B, OUT), jnp.float32),
    )(x, gamma.reshape(1, D_IN), beta.reshape(1, D_IN),
      mm.reshape(1, D_IN), mv.reshape(1, D_IN),
      W1, b1.reshape(1, H1), W2, b2.reshape(1, H2), W3, b3.reshape(1, OUT))


def kernel(inputs, embed_tables, gamma, beta, moving_mean, moving_var,
           W1, b1, W2, b2, W3, b3):
    sparse_idx = inputs[:, N_DENSE:].astype(jnp.int32)
    flat_idx = sparse_idx + jnp.arange(N_SPARSE, dtype=jnp.int32) * VOCAB
    idx3d = flat_idx.reshape(NW, NG, GATHER_ROWS)
    table2d = embed_tables.reshape(N_SPARSE * VOCAB, EMBED)

    inter = _sc_pool(idx3d, table2d)
    x = jnp.concatenate([inter, inputs[:, :N_DENSE]], axis=1)
    return _mlp(x, gamma, beta, moving_mean, moving_var,
                W1, b1, W2, b2, W3, b3)


# speedup vs baseline: 1.1408x; 1.1408x over previous
import jax
import jax.numpy as jnp
from jax import lax
from jax.experimental import pallas as pl
from jax.experimental.pallas import tpu as pltpu
from jax.experimental.pallas import tpu_sc as plsc

N_DENSE = 13
N_SPARSE = 26
VOCAB = 100000
EMBED = 16
B = 16384
H1, H2, OUT = 128, 64, 1
D_IN = EMBED + N_DENSE

NC, NS = 2, 16
NW = NC * NS
SPW = B // NW
ROWS_PW = SPW * N_SPARSE
GATHER_ROWS = 128
NG = ROWS_PW // GATHER_ROWS
CH_G = 13
SPC = CH_G * GATHER_ROWS // N_SPARSE
NCHUNK = NG // CH_G


def _sc_pool_body(idx_hbm, table_hbm, inter_hbm,
                  idx_v, rows0, rows1, out_v, sem0, sem1):
    cid = lax.axis_index("c")
    sid = lax.axis_index("s")
    wid = sid * NC + cid

    pltpu.sync_copy(idx_hbm.at[wid], idx_v)

    def fire(c, rows, sem):
        descs = []
        for j in range(CH_G):
            d = pltpu.async_copy(
                table_hbm.at[idx_v.at[c * CH_G + j]],
                rows.at[pl.ds(j * GATHER_ROWS, GATHER_ROWS)],
                sem)
            descs.append(d)
        return descs

    def accumulate(c, rows):
        def body(b, carry):
            base = b * N_SPARSE
            v = rows[base]
            s = v
            q = v * v
            for f in range(1, N_SPARSE):
                v = rows[base + f]
                s = s + v
                q = q + v * v
            out_v[c * SPC + b] = 0.5 * (s * s - q)
            return carry
        lax.fori_loop(0, SPC, body, 0)

    bufs = (rows0, rows1)
    sems = (sem0, sem1)
    descs = fire(0, bufs[0], sems[0])
    for c in range(NCHUNK):
        nxt = None
        if c + 1 < NCHUNK:
            nxt = fire(c + 1, bufs[(c + 1) % 2], sems[(c + 1) % 2])
        for d in descs:
            d.wait()
        accumulate(c, bufs[c % 2])
        descs = nxt

    pltpu.sync_copy(out_v, inter_hbm.at[pl.ds(wid * SPW, SPW)])


@jax.jit
def _sc_pool(idx3d, table2d):
    mesh = plsc.VectorSubcoreMesh(core_axis_name="c", subcore_axis_name="s")
    return pl.kernel(
        _sc_pool_body,
        out_type=jax.ShapeDtypeStruct((B, EMBED), jnp.float32),
        mesh=mesh,
        compiler_params=pltpu.CompilerParams(use_tc_tiling_on_sc=False),
        scratch_types=[
            pltpu.VMEM((NG, GATHER_ROWS), jnp.int32),
            pltpu.VMEM((CH_G * GATHER_ROWS, EMBED), jnp.float32),
            pltpu.VMEM((CH_G * GATHER_ROWS, EMBED), jnp.float32),
            pltpu.VMEM((SPW, EMBED), jnp.float32),
            pltpu.SemaphoreType.DMA,
            pltpu.SemaphoreType.DMA,
        ],
    )(idx3d, table2d)


BS = 2048


def _mlp_body(x_ref, g_ref, be_ref, mm_ref, mv_ref,
              w1_ref, b1_ref, w2_ref, b2_ref, w3_ref, b3_ref, o_ref):
    x = x_ref[...]
    inv = lax.rsqrt(mv_ref[...] + 1e-3)
    x = (x - mm_ref[...]) * inv * g_ref[...] + be_ref[...]
    h = jnp.dot(x, w1_ref[...], preferred_element_type=jnp.float32) + b1_ref[...]
    h = jnp.maximum(h, 0.0)
    h = jnp.dot(h, w2_ref[...], preferred_element_type=jnp.float32) + b2_ref[...]
    h = jnp.maximum(h, 0.0)
    o = jnp.dot(h, w3_ref[...], preferred_element_type=jnp.float32) + b3_ref[...]
    o_ref[...] = jax.nn.sigmoid(o)


@jax.jit
def _mlp(x, gamma, beta, mm, mv, W1, b1, W2, b2, W3, b3):
    full = lambda shape: pl.BlockSpec(shape, lambda i: (0, 0))
    return pl.pallas_call(
        _mlp_body,
        grid=(B // BS,),
        in_specs=[
            pl.BlockSpec((BS, D_IN), lambda i: (i, 0)),
            full((1, D_IN)), full((1, D_IN)), full((1, D_IN)), full((1, D_IN)),
            full((D_IN, H1)), full((1, H1)),
            full((H1, H2)), full((1, H2)),
            full((H2, OUT)), full((1, OUT)),
        ],
        out_specs=pl.BlockSpec((BS, OUT), lambda i: (i, 0)),
        out_shape=jax.ShapeDtypeStruct((B, OUT), jnp.float32),
    )(x, gamma.reshape(1, D_IN), beta.reshape(1, D_IN),
      mm.reshape(1, D_IN), mv.reshape(1, D_IN),
      W1, b1.reshape(1, H1), W2, b2.reshape(1, H2), W3, b3.reshape(1, OUT))


def kernel(inputs, embed_tables, gamma, beta, moving_mean, moving_var,
           W1, b1, W2, b2, W3, b3):
    sparse_idx = inputs[:, N_DENSE:].astype(jnp.int32)
    flat_idx = sparse_idx + jnp.arange(N_SPARSE, dtype=jnp.int32) * VOCAB
    idx3d = flat_idx.reshape(NW, NG, GATHER_ROWS)
    table2d = embed_tables.reshape(N_SPARSE * VOCAB, EMBED)

    inter = _sc_pool(idx3d, table2d)
    x = jnp.concatenate([inter, inputs[:, :N_DENSE]], axis=1)
    return _mlp(x, gamma, beta, moving_mean, moving_var,
                W1, b1, W2, b2, W3, b3)


# speedup vs baseline: 4.5025x; 3.9467x over previous
import jax
import jax.numpy as jnp
from jax import lax
from jax.experimental import pallas as pl
from jax.experimental.pallas import tpu as pltpu
from jax.experimental.pallas import tpu_sc as plsc

N_DENSE = 13
N_SPARSE = 26
VOCAB = 100000
EMBED = 16
B = 16384
H1, H2, OUT = 128, 64, 1
D_IN = EMBED + N_DENSE

NC, NS = 2, 16
NW = NC * NS
NHALF = 2
HB = B // NHALF
LANES = 16


def _sc_pool_body(planes_hbm, idx_hbm, inter_t_hbm,
                  plane_v, idx_v, s_v, q_v):
    cid = lax.axis_index("c")
    sid = lax.axis_index("s")
    wid = sid * NC + cid
    e = wid // NHALF
    h = wid % NHALF

    zero = jnp.zeros((LANES,), jnp.float32)

    def zbody(i, carry):
        s_v[pl.ds(i * LANES, LANES)] = zero
        q_v[pl.ds(i * LANES, LANES)] = zero
        return carry
    lax.fori_loop(0, HB // LANES, zbody, 0)

    def per_field(f, carry):
        pltpu.sync_copy(planes_hbm.at[f * EMBED + e], plane_v)
        pltpu.sync_copy(idx_hbm.at[f, h], idx_v)

        def gbody(i, c2):
            sl = pl.ds(i * LANES, LANES)
            ids = idx_v[sl]
            v = plsc.load_gather(plane_v, [ids])
            plsc.addupdate(s_v.at[sl], v)
            plsc.addupdate(q_v.at[sl], v * v)
            return c2
        lax.fori_loop(0, HB // LANES, gbody, 0)
        return carry
    lax.fori_loop(0, N_SPARSE, per_field, 0)

    def finbody(i, carry):
        sl = pl.ds(i * LANES, LANES)
        s = s_v[sl]
        q = q_v[sl]
        s_v[sl] = 0.5 * (s * s - q)
        return carry
    lax.fori_loop(0, HB // LANES, finbody, 0)

    pltpu.sync_copy(s_v, inter_t_hbm.at[e, pl.ds(h * HB, HB)])


@jax.jit
def _sc_pool(planes, idx_t):
    mesh = plsc.VectorSubcoreMesh(core_axis_name="c", subcore_axis_name="s")
    return pl.kernel(
        _sc_pool_body,
        out_type=jax.ShapeDtypeStruct((EMBED, B), jnp.float32),
        mesh=mesh,
        compiler_params=pltpu.CompilerParams(needs_layout_passes=False),
        scratch_types=[
            pltpu.VMEM((VOCAB,), jnp.float32),
            pltpu.VMEM((HB,), jnp.int32),
            pltpu.VMEM((HB,), jnp.float32),
            pltpu.VMEM((HB,), jnp.float32),
        ],
    )(planes, idx_t)


BS = 2048


def _mlp_body(x_ref, g_ref, be_ref, mm_ref, mv_ref,
              w1_ref, b1_ref, w2_ref, b2_ref, w3_ref, b3_ref, o_ref):
    x = x_ref[...]
    inv = lax.rsqrt(mv_ref[...] + 1e-3)
    x = (x - mm_ref[...]) * inv * g_ref[...] + be_ref[...]
    h = jnp.dot(x, w1_ref[...], preferred_element_type=jnp.float32) + b1_ref[...]
    h = jnp.maximum(h, 0.0)
    h = jnp.dot(h, w2_ref[...], preferred_element_type=jnp.float32) + b2_ref[...]
    h = jnp.maximum(h, 0.0)
    o = jnp.dot(h, w3_ref[...], preferred_element_type=jnp.float32) + b3_ref[...]
    o_ref[...] = jax.nn.sigmoid(o)


@jax.jit
def _mlp(x, gamma, beta, mm, mv, W1, b1, W2, b2, W3, b3):
    full = lambda shape: pl.BlockSpec(shape, lambda i: (0, 0))
    return pl.pallas_call(
        _mlp_body,
        grid=(B // BS,),
        in_specs=[
            pl.BlockSpec((BS, D_IN), lambda i: (i, 0)),
            full((1, D_IN)), full((1, D_IN)), full((1, D_IN)), full((1, D_IN)),
            full((D_IN, H1)), full((1, H1)),
            full((H1, H2)), full((1, H2)),
            full((H2, OUT)), full((1, OUT)),
        ],
        out_specs=pl.BlockSpec((BS, OUT), lambda i: (i, 0)),
        out_shape=jax.ShapeDtypeStruct((B, OUT), jnp.float32),
    )(x, gamma.reshape(1, D_IN), beta.reshape(1, D_IN),
      mm.reshape(1, D_IN), mv.reshape(1, D_IN),
      W1, b1.reshape(1, H1), W2, b2.reshape(1, H2), W3, b3.reshape(1, OUT))


def kernel(inputs, embed_tables, gamma, beta, moving_mean, moving_var,
           W1, b1, W2, b2, W3, b3):
    planes = embed_tables.transpose(0, 2, 1).reshape(N_SPARSE * EMBED, VOCAB)
    sparse_idx = inputs[:, N_DENSE:].astype(jnp.int32)
    idx_t = sparse_idx.T.reshape(N_SPARSE, NHALF, HB)

    inter_t = _sc_pool(planes, idx_t)
    x = jnp.concatenate([inter_t.T, inputs[:, :N_DENSE]], axis=1)
    return _mlp(x, gamma, beta, moving_mean, moving_var,
                W1, b1, W2, b2, W3, b3)
